# baseline (device time: 22691 ns/iter reference)
import jax
import jax.numpy as jnp
from jax import lax
from jax.experimental import pallas as pl
from jax.experimental.pallas import tpu as pltpu

N_DEV = 16
NZ = 4
NQ = 4
NS = 2

COMM_DT = jnp.bfloat16


def kernel(x, w_mat):
    m, k_loc = x.shape
    _, n = w_mat.shape
    m_per = m // N_DEV
    mq = NZ * m_per
    ms = mq // NS
    n2 = n // 2
    f32 = jnp.float32

    def body(x_ref, w_ref, out_ref, xp, fcomm, bcomm, rbuf, sbuf, pbuf,
             fss, frs, bss, brs, pss, prs):
        my = lax.axis_index("i")
        q = lax.rem(my, NQ)
        t = my // NQ
        base = my - q
        right = base + lax.rem(q + 1, NQ)
        left = base + lax.rem(q + 3, NQ)

        bar = pltpu.get_barrier_semaphore()
        for nbr in (left, right):
            pl.semaphore_signal(bar, inc=1, device_id=(nbr,),
                                device_id_type=pl.DeviceIdType.MESH)

        for T in range(NZ):
            @pl.when(t == T)
            def _(T=T):
                for tp in range(NZ):
                    if tp != T:
                        pl.semaphore_signal(
                            bar, inc=1, device_id=(q + NQ * tp,),
                            device_id_type=pl.DeviceIdType.MESH)

        for r in range(NQ):
            for j in range(NZ):
                xp[pl.ds(r * mq + j * m_per, m_per), :] = (
                    x_ref[pl.ds((NQ * j + r) * m_per, m_per), :]
                )

        pl.semaphore_wait(bar, 2 + (NZ - 1))

        def quarter(r):
            return jnp.dot(
                xp[pl.ds(r * mq, mq), :], w_ref[:, :],
                preferred_element_type=f32,
            )

        def p1_rdma(comm, h, s, ssem, rsem, target):
            r = pltpu.make_async_remote_copy(
                src_ref=comm.at[h, s], dst_ref=comm.at[h + 1, s],
                send_sem=ssem.at[h, s], recv_sem=rsem.at[h, s],
                device_id=(target,), device_id_type=pl.DeviceIdType.MESH,
            )
            r.start()
            return r

        qm1 = lax.rem(q + 3, NQ)
        qp1 = lax.rem(q + 1, NQ)
        init_f = quarter(qm1)
        init_b = quarter(qp1)
        p1_sends = []
        frd = [None, None]
        brd = [None, None]
        for s in range(NS):
            r0, r1 = s * ms, (s + 1) * ms
            fcomm[0, s] = init_f[r0:r1, :n2].astype(COMM_DT)
            frd[s] = p1_rdma(fcomm, 0, s, fss, frs, right)
            bcomm[0, s] = init_b[r0:r1, n2:].astype(COMM_DT)
            brd[s] = p1_rdma(bcomm, 0, s, bss, brs, left)
        p1_sends += frd + brd

        for h in range(NQ - 1):
            qf = lax.rem(q + 2 * NQ - 2 - h, NQ)
            qb = lax.rem(q + 2 + h, NQ)
            pf = quarter(qf)
            pb = pf if h != 1 else quarter(qb)
            nfrd = [None, None]
            nbrd = [None, None]
            for s in range(NS):
                r0, r1 = s * ms, (s + 1) * ms
                frd[s].wait_recv()
                if h < NQ - 2:
                    fcomm[h + 1, s] = (
                        fcomm[h + 1, s].astype(f32) + pf[r0:r1, :n2]
                    ).astype(COMM_DT)
                    nfrd[s] = p1_rdma(fcomm, h + 1, s, fss, frs, right)
                else:
                    rbuf[r0:r1, :n2] = fcomm[h + 1, s].astype(f32) + pf[r0:r1, :n2]
                brd[s].wait_recv()
                if h < NQ - 2:
                    bcomm[h + 1, s] = (
                        bcomm[h + 1, s].astype(f32) + pb[r0:r1, n2:]
                    ).astype(COMM_DT)
                    nbrd[s] = p1_rdma(bcomm, h + 1, s, bss, brs, left)
                else:
                    rbuf[r0:r1, n2:] = bcomm[h + 1, s].astype(f32) + pb[r0:r1, n2:]
            if h < NQ - 2:
                frd, brd = nfrd, nbrd
                p1_sends += nfrd + nbrd

        def rb(d):
            return rbuf[d * m_per:(d + 1) * m_per, :]

        for T in range(NZ):
            @pl.when(t == T)
            def _(T=T):
                others = [tp for tp in range(NZ) if tp != T]
                sends = []
                for tp in sorted(others, key=lambda v: -abs(v - T)):
                    sbuf[tp] = rb(tp).astype(COMM_DT)
                    r = pltpu.make_async_remote_copy(
                        src_ref=sbuf.at[tp], dst_ref=pbuf.at[T],
                        send_sem=pss.at[tp], recv_sem=prs.at[T],
                        device_id=(q + NQ * tp,),
                        device_id_type=pl.DeviceIdType.MESH,
                    )
                    r.start()
                    sends.append(r)
                acc = rb(T)
                for tp in sorted(others, key=lambda v: abs(v - T)):
                    rr = pltpu.make_async_remote_copy(
                        src_ref=sbuf.at[tp], dst_ref=pbuf.at[tp],
                        send_sem=pss.at[tp], recv_sem=prs.at[tp],
                        device_id=(my,),
                        device_id_type=pl.DeviceIdType.MESH,
                    )
                    rr.wait_recv()
                    acc = acc + pbuf[tp].astype(f32)
                out_ref[:, :] = acc
                for s_ in sends:
                    s_.wait_send()

        for s_ in p1_sends:
            s_.wait_send()

    return pl.pallas_call(
        body,
        out_shape=jax.ShapeDtypeStruct((m_per, n), jnp.float32),
        in_specs=[
            pl.BlockSpec(memory_space=pltpu.VMEM),
            pl.BlockSpec(memory_space=pltpu.VMEM),
        ],
        out_specs=pl.BlockSpec(memory_space=pltpu.VMEM),
        scratch_shapes=[
            pltpu.VMEM((NQ * NZ * m_per, k_loc), jnp.float32),
            pltpu.VMEM((NQ, NS, NZ * m_per // NS, n2), COMM_DT),
            pltpu.VMEM((NQ, NS, NZ * m_per // NS, n2), COMM_DT),
            pltpu.VMEM((NZ * m_per, n), jnp.float32),
            pltpu.VMEM((NZ, m_per, n), COMM_DT),
            pltpu.VMEM((NZ, m_per, n), COMM_DT),
            pltpu.SemaphoreType.DMA((NQ - 1, NS)),
            pltpu.SemaphoreType.DMA((NQ - 1, NS)),
            pltpu.SemaphoreType.DMA((NQ - 1, NS)),
            pltpu.SemaphoreType.DMA((NQ - 1, NS)),
            pltpu.SemaphoreType.DMA((NZ,)),
            pltpu.SemaphoreType.DMA((NZ,)),
        ],
        compiler_params=pltpu.CompilerParams(collective_id=0),
    )(x, w_mat)
